# baseline (device time: 30181 ns/iter reference)
import jax
import jax.numpy as jnp
from jax import lax
from jax.experimental import pallas as pl
from jax.experimental.pallas import tpu as pltpu

N_DEV = 4
B, SQ, DM = 2, 128, 512
HQ, DH = 4, 64
SKV_SH = 128
GQ = 2
BLK = 64


def kernel(x, Wq, K_ext, V_ext, Wo):
    xb = x.astype(jnp.bfloat16)
    wqb = Wq.astype(jnp.bfloat16)
    wob = Wo.astype(jnp.bfloat16)
    kt = jnp.transpose(K_ext.astype(jnp.bfloat16), (0, 2, 1, 3))
    vt = jnp.transpose(V_ext.astype(jnp.bfloat16), (0, 2, 1, 3))

    def body(x_ref, wq_ref, k_ref, v_ref, wo_ref, out_ref,
             k0_buf, v0_buf, k2_buf, v2_buf,
             send_sems, recv_sems, credit_sem):
        my = lax.axis_index("i")

        @pl.when(my == 0)
        def _():
            k0_buf[...] = k_ref[...]
            v0_buf[...] = v_ref[...]

        @pl.when(my == 2)
        def _():
            k2_buf[...] = k_ref[...]
            v2_buf[...] = v_ref[...]

        def bcast(buf, dsts, send_base, recv_idx):
            for j, dst in enumerate(dsts):
                pltpu.make_async_remote_copy(
                    src_ref=buf,
                    dst_ref=buf,
                    send_sem=send_sems.at[send_base + j],
                    recv_sem=recv_sems.at[recv_idx],
                    device_id=(dst,),
                    device_id_type=pl.DeviceIdType.MESH,
                ).start()

        @pl.when(my == 0)
        def _():
            bcast(k0_buf, (1, 2, 3), 0, 0)
            bcast(v0_buf, (1, 2, 3), 3, 1)

        @pl.when(my == 2)
        def _():
            bcast(k2_buf, (0, 1, 3), 0, 2)
            bcast(v2_buf, (0, 1, 3), 3, 3)

        q = [
            jnp.dot(x_ref[b], wq_ref[...], preferred_element_type=jnp.float32)
            for b in range(B)
        ]

        def wait_recv(buf, recv_idx):
            pltpu.make_async_remote_copy(
                src_ref=buf,
                dst_ref=buf,
                send_sem=send_sems.at[0],
                recv_sem=recv_sems.at[recv_idx],
                device_id=(0,),
                device_id_type=pl.DeviceIdType.MESH,
            ).wait_recv()

        @pl.when(my != 0)
        def _():
            wait_recv(k0_buf, 0)
            wait_recv(v0_buf, 1)

        @pl.when(my != 2)
        def _():
            wait_recv(k2_buf, 2)
            wait_recv(v2_buf, 3)

        @pl.when((my == 0) | (my == 2))
        def _():
            for j in range(6):
                pltpu.make_async_remote_copy(
                    src_ref=k0_buf,
                    dst_ref=k0_buf,
                    send_sem=send_sems.at[j],
                    recv_sem=recv_sems.at[0],
                    device_id=(0,),
                    device_id_type=pl.DeviceIdType.MESH,
                ).wait_send()

        for b in range(B):
            for g in range(GQ):
                ctx_heads = []
                for h in range(HQ):
                    qh = q[b][g * BLK:(g + 1) * BLK,
                              h * DH:(h + 1) * DH].astype(jnp.bfloat16)
                    k0h = k0_buf[b, h, pl.ds(g * BLK, BLK), :]
                    k2h = k2_buf[b, h, pl.ds(g * BLK, BLK), :]
                    s0 = lax.dot_general(
                        qh, k0h, (((1,), (1,)), ((), ())),
                        preferred_element_type=jnp.float32) * 0.125
                    s1 = lax.dot_general(
                        qh, k2h, (((1,), (1,)), ((), ())),
                        preferred_element_type=jnp.float32) * 0.125
                    m = jnp.maximum(
                        jnp.max(s0, axis=1, keepdims=True),
                        jnp.max(s1, axis=1, keepdims=True))
                    e0 = jnp.exp(s0 - m)
                    e1 = jnp.exp(s1 - m)
                    denom = (jnp.sum(e0, axis=1, keepdims=True)
                             + jnp.sum(e1, axis=1, keepdims=True))
                    w0 = (e0 / denom).astype(jnp.bfloat16)
                    w1 = (e1 / denom).astype(jnp.bfloat16)
                    v0h = v0_buf[b, h, pl.ds(g * BLK, BLK), :]
                    v2h = v2_buf[b, h, pl.ds(g * BLK, BLK), :]
                    ctx_h = (
                        jnp.dot(w0, v0h, preferred_element_type=jnp.float32)
                        + jnp.dot(w1, v2h, preferred_element_type=jnp.float32)
                    )
                    ctx_heads.append(ctx_h)
                ctx = jnp.concatenate(ctx_heads, axis=1).astype(jnp.bfloat16)
                out_bg = jnp.dot(ctx, wo_ref[...],
                                 preferred_element_type=jnp.float32)
                out_ref[b, pl.ds(g * BLK, BLK), :] = out_bg

        @pl.when(my != 0)
        def _():
            pl.semaphore_signal(credit_sem, inc=1, device_id=(0,),
                                device_id_type=pl.DeviceIdType.MESH)

        @pl.when(my != 2)
        def _():
            pl.semaphore_signal(credit_sem, inc=1, device_id=(2,),
                                device_id_type=pl.DeviceIdType.MESH)

        @pl.when((my == 0) | (my == 2))
        def _():
            pl.semaphore_wait(credit_sem, 3)

    return pl.pallas_call(
        body,
        out_shape=jax.ShapeDtypeStruct((B, SQ, DM), jnp.float32),
        in_specs=[pl.BlockSpec(memory_space=pltpu.VMEM)] * 5,
        out_specs=pl.BlockSpec(memory_space=pltpu.VMEM),
        scratch_shapes=[
            pltpu.VMEM((B, HQ, SKV_SH, DH), jnp.bfloat16),
            pltpu.VMEM((B, HQ, SKV_SH, DH), jnp.bfloat16),
            pltpu.VMEM((B, HQ, SKV_SH, DH), jnp.bfloat16),
            pltpu.VMEM((B, HQ, SKV_SH, DH), jnp.bfloat16),
            pltpu.SemaphoreType.DMA((6,)),
            pltpu.SemaphoreType.DMA((4,)),
            pltpu.SemaphoreType.REGULAR,
        ],
    )(xb, wqb, kt, vt, wob)


# device time: 23689 ns/iter; 1.2741x vs baseline; 1.2741x over previous
import jax
import jax.numpy as jnp
from jax import lax
from jax.experimental import pallas as pl
from jax.experimental.pallas import tpu as pltpu

N_DEV = 4
B, SQ, DM = 2, 128, 512
HQ, DH = 4, 64
GQ = 2
BLK = 64


def kernel(x, Wq, K_ext, V_ext, Wo):
    xb = x.astype(jnp.bfloat16)
    wqb = Wq.astype(jnp.bfloat16)
    wob = Wo.astype(jnp.bfloat16)
    kt = jnp.transpose(K_ext.astype(jnp.bfloat16), (0, 2, 1, 3)).reshape(
        B, HQ, GQ, BLK, DH)
    vt = jnp.transpose(V_ext.astype(jnp.bfloat16), (0, 2, 1, 3)).reshape(
        B, HQ, GQ, BLK, DH)

    def body(x_ref, wq_ref, k_ref, v_ref, wo_ref, out_ref,
             kv_buf, send_sems, recv_sems):
        my = lax.axis_index("i")

        barrier_sem = pltpu.get_barrier_semaphore()

        @pl.when(my != 0)
        def _():
            pl.semaphore_signal(barrier_sem, inc=1, device_id=(0,),
                                device_id_type=pl.DeviceIdType.MESH)

        @pl.when(my != 2)
        def _():
            pl.semaphore_signal(barrier_sem, inc=1, device_id=(2,),
                                device_id_type=pl.DeviceIdType.MESH)

        def stage_and_send(slot, dsts):
            kv_buf[0, :, :, :, slot] = k_ref[...]
            kv_buf[1, :, :, :, slot] = v_ref[...]
            pl.semaphore_wait(barrier_sem, 3)
            for j, dst in enumerate(dsts):
                pltpu.make_async_remote_copy(
                    src_ref=kv_buf.at[:, :, :, :, slot],
                    dst_ref=kv_buf.at[:, :, :, :, slot],
                    send_sem=send_sems.at[j],
                    recv_sem=recv_sems.at[slot],
                    device_id=(dst,),
                    device_id_type=pl.DeviceIdType.MESH,
                ).start()

        @pl.when(my == 0)
        def _():
            stage_and_send(0, (1, 2, 3))

        @pl.when(my == 2)
        def _():
            stage_and_send(1, (0, 1, 3))

        q = [
            jnp.dot(x_ref[b], wq_ref[...], preferred_element_type=jnp.float32)
            for b in range(B)
        ]

        def wait_recv(slot):
            pltpu.make_async_remote_copy(
                src_ref=kv_buf.at[:, :, :, :, slot],
                dst_ref=kv_buf.at[:, :, :, :, slot],
                send_sem=send_sems.at[0],
                recv_sem=recv_sems.at[slot],
                device_id=(0,),
                device_id_type=pl.DeviceIdType.MESH,
            ).wait_recv()

        @pl.when(my != 0)
        def _():
            wait_recv(0)

        @pl.when(my != 2)
        def _():
            wait_recv(1)

        @pl.when((my == 0) | (my == 2))
        def _():
            for j in range(3):
                pltpu.make_async_remote_copy(
                    src_ref=kv_buf.at[:, :, :, :, 0],
                    dst_ref=kv_buf.at[:, :, :, :, 0],
                    send_sem=send_sems.at[j],
                    recv_sem=recv_sems.at[0],
                    device_id=(0,),
                    device_id_type=pl.DeviceIdType.MESH,
                ).wait_send()

        for b in range(B):
            for g in range(GQ):
                ctx_heads = []
                for h in range(HQ):
                    qh = q[b][g * BLK:(g + 1) * BLK,
                              h * DH:(h + 1) * DH].astype(jnp.bfloat16)
                    kcat = kv_buf[0, b, h, g].reshape(2 * BLK, DH)
                    vcat = kv_buf[1, b, h, g].reshape(2 * BLK, DH)
                    s = lax.dot_general(
                        qh, kcat, (((1,), (1,)), ((), ())),
                        preferred_element_type=jnp.float32) * 0.125
                    m = jnp.max(s, axis=1, keepdims=True)
                    e = jnp.exp(s - m)
                    w = (e / jnp.sum(e, axis=1, keepdims=True)).astype(
                        jnp.bfloat16)
                    ctx_heads.append(
                        jnp.dot(w, vcat, preferred_element_type=jnp.float32))
                ctx = jnp.concatenate(ctx_heads, axis=1).astype(jnp.bfloat16)
                out_ref[b, pl.ds(g * BLK, BLK), :] = jnp.dot(
                    ctx, wo_ref[...], preferred_element_type=jnp.float32)

    return pl.pallas_call(
        body,
        out_shape=jax.ShapeDtypeStruct((B, SQ, DM), jnp.float32),
        in_specs=[pl.BlockSpec(memory_space=pltpu.VMEM)] * 5,
        out_specs=pl.BlockSpec(memory_space=pltpu.VMEM),
        scratch_shapes=[
            pltpu.VMEM((2, B, HQ, GQ, 2, BLK, DH), jnp.bfloat16),
            pltpu.SemaphoreType.DMA((3,)),
            pltpu.SemaphoreType.DMA((2,)),
        ],
        compiler_params=pltpu.CompilerParams(collective_id=0),
    )(xb, wqb, kt, vt, wob)


# device time: 14893 ns/iter; 2.0265x vs baseline; 1.5906x over previous
import jax
import jax.numpy as jnp
from jax import lax
from jax.experimental import pallas as pl
from jax.experimental.pallas import tpu as pltpu

N_DEV = 4
B, SQ, DM = 2, 128, 512
HQ, DH = 4, 64
GQ = 2
BLK = 64
PK = DH + 1


def kernel(x, Wq, K_ext, V_ext, Wo):
    xb = x.astype(jnp.bfloat16)
    wqb = Wq.astype(jnp.bfloat16)
    wob = Wo.astype(jnp.bfloat16)
    kt = jnp.transpose(K_ext.astype(jnp.bfloat16), (0, 2, 1, 3)).reshape(
        B, HQ, GQ, BLK, DH)
    vt = jnp.transpose(V_ext.astype(jnp.bfloat16), (0, 2, 1, 3)).reshape(
        B, HQ, GQ, BLK, DH)

    def body(x_ref, wq_ref, k_ref, v_ref, wo_ref, out_ref,
             cl_buf, send_sems, recv_sems):
        my = lax.axis_index("i")

        barrier_sem = pltpu.get_barrier_semaphore()

        @pl.when(my != 0)
        def _():
            pl.semaphore_signal(barrier_sem, inc=1, device_id=(0,),
                                device_id_type=pl.DeviceIdType.MESH)

        @pl.when(my != 2)
        def _():
            pl.semaphore_signal(barrier_sem, inc=1, device_id=(2,),
                                device_id_type=pl.DeviceIdType.MESH)

        def partial_attn(slot, dsts):
            q = [
                jnp.dot(x_ref[b], wq_ref[...],
                        preferred_element_type=jnp.float32) * 0.125
                for b in range(B)
            ]
            for g in range(GQ):
                for b in range(B):
                    for h in range(HQ):
                        qh = q[b][g * BLK:(g + 1) * BLK,
                                  h * DH:(h + 1) * DH].astype(jnp.bfloat16)
                        kh = k_ref[b, h, g]
                        vh = v_ref[b, h, g]
                        s = lax.dot_general(
                            qh, kh, (((1,), (1,)), ((), ())),
                            preferred_element_type=jnp.float32)
                        e = jnp.exp(s)
                        l = jnp.sum(e, axis=1, keepdims=True)
                        c = jnp.dot(e.astype(jnp.bfloat16), vh,
                                    preferred_element_type=jnp.float32)
                        cl_buf[slot, b, g, h] = jnp.concatenate(
                            [c, l], axis=1).astype(jnp.bfloat16)
                if g == 0:
                    pl.semaphore_wait(barrier_sem, 3)
                for j, dst in enumerate(dsts):
                    pltpu.make_async_remote_copy(
                        src_ref=cl_buf.at[slot, :, g],
                        dst_ref=cl_buf.at[slot, :, g],
                        send_sem=send_sems.at[g * 3 + j],
                        recv_sem=recv_sems.at[slot, g],
                        device_id=(dst,),
                        device_id_type=pl.DeviceIdType.MESH,
                    ).start()

        @pl.when(my == 0)
        def _():
            partial_attn(0, (1, 2, 3))

        @pl.when(my == 2)
        def _():
            partial_attn(1, (0, 1, 3))

        def wait_recv(slot, g):
            pltpu.make_async_remote_copy(
                src_ref=cl_buf.at[slot, :, g],
                dst_ref=cl_buf.at[slot, :, g],
                send_sem=send_sems.at[0],
                recv_sem=recv_sems.at[slot, g],
                device_id=(0,),
                device_id_type=pl.DeviceIdType.MESH,
            ).wait_recv()

        for g in range(GQ):
            @pl.when(my != 0)
            def _():
                wait_recv(0, g)

            @pl.when(my != 2)
            def _():
                wait_recv(1, g)

            for b in range(B):
                ctx_heads = []
                for h in range(HQ):
                    p0 = cl_buf[0, b, g, h].astype(jnp.float32)
                    p2 = cl_buf[1, b, g, h].astype(jnp.float32)
                    p = p0 + p2
                    ctx_heads.append(p[:, :DH] / p[:, DH:PK])
                ctx = jnp.concatenate(ctx_heads, axis=1).astype(jnp.bfloat16)
                out_ref[b, pl.ds(g * BLK, BLK), :] = jnp.dot(
                    ctx, wo_ref[...], preferred_element_type=jnp.float32)

        @pl.when((my == 0) | (my == 2))
        def _():
            for j in range(6):
                pltpu.make_async_remote_copy(
                    src_ref=cl_buf.at[0, :, 0],
                    dst_ref=cl_buf.at[0, :, 0],
                    send_sem=send_sems.at[j],
                    recv_sem=recv_sems.at[0, 0],
                    device_id=(0,),
                    device_id_type=pl.DeviceIdType.MESH,
                ).wait_send()

    return pl.pallas_call(
        body,
        out_shape=jax.ShapeDtypeStruct((B, SQ, DM), jnp.float32),
        in_specs=[pl.BlockSpec(memory_space=pltpu.VMEM)] * 5,
        out_specs=pl.BlockSpec(memory_space=pltpu.VMEM),
        scratch_shapes=[
            pltpu.VMEM((2, B, GQ, HQ, BLK, PK), jnp.bfloat16),
            pltpu.SemaphoreType.DMA((6,)),
            pltpu.SemaphoreType.DMA((2, 2)),
        ],
        compiler_params=pltpu.CompilerParams(collective_id=0),
    )(xb, wqb, kt, vt, wob)


# device time: 14070 ns/iter; 2.1451x vs baseline; 1.0585x over previous
import jax
import jax.numpy as jnp
from jax import lax
from jax.experimental import pallas as pl
from jax.experimental.pallas import tpu as pltpu

N_DEV = 4
B, SQ, DM = 2, 128, 512
HQ, DH = 4, 64
GQ = 2
BLK = 64
PK = DH + 1


def kernel(x, Wq, K_ext, V_ext, Wo):

    def body(x_ref, wq_ref, k_ref, v_ref, wo_ref, out_ref,
             cl_buf, send_sems, recv_sems):
        my = lax.axis_index("i")

        barrier_sem = pltpu.get_barrier_semaphore()

        @pl.when(my != 0)
        def _():
            pl.semaphore_signal(barrier_sem, inc=1, device_id=(0,),
                                device_id_type=pl.DeviceIdType.MESH)

        @pl.when(my != 2)
        def _():
            pl.semaphore_signal(barrier_sem, inc=1, device_id=(2,),
                                device_id_type=pl.DeviceIdType.MESH)

        def partial_attn(slot, dsts):
            wqb = wq_ref[...].astype(jnp.bfloat16)
            q = [
                jnp.dot(x_ref[b].astype(jnp.bfloat16), wqb,
                        preferred_element_type=jnp.float32) * 0.125
                for b in range(B)
            ]
            for g in range(GQ):
                for b in range(B):
                    for h in range(HQ):
                        qh = q[b][g * BLK:(g + 1) * BLK,
                                  h * DH:(h + 1) * DH].astype(jnp.bfloat16)
                        kh = k_ref[b, pl.ds(g * BLK, BLK), h, :].astype(
                            jnp.bfloat16)
                        vh = v_ref[b, pl.ds(g * BLK, BLK), h, :].astype(
                            jnp.bfloat16)
                        s = lax.dot_general(
                            qh, kh, (((1,), (1,)), ((), ())),
                            preferred_element_type=jnp.float32)
                        e = jnp.exp(s)
                        l = jnp.sum(e, axis=1, keepdims=True)
                        c = jnp.dot(e.astype(jnp.bfloat16), vh,
                                    preferred_element_type=jnp.float32)
                        cl_buf[slot, b, g, h] = jnp.concatenate(
                            [c, l], axis=1).astype(jnp.bfloat16)
                if g == 0:
                    pl.semaphore_wait(barrier_sem, 3)
                for j, dst in enumerate(dsts):
                    pltpu.make_async_remote_copy(
                        src_ref=cl_buf.at[slot, :, g],
                        dst_ref=cl_buf.at[slot, :, g],
                        send_sem=send_sems.at[g * 3 + j],
                        recv_sem=recv_sems.at[slot, g],
                        device_id=(dst,),
                        device_id_type=pl.DeviceIdType.MESH,
                    ).start()

        @pl.when(my == 0)
        def _():
            partial_attn(0, (1, 2, 3))

        @pl.when(my == 2)
        def _():
            partial_attn(1, (0, 1, 3))

        def wait_recv(slot, g):
            pltpu.make_async_remote_copy(
                src_ref=cl_buf.at[slot, :, g],
                dst_ref=cl_buf.at[slot, :, g],
                send_sem=send_sems.at[0],
                recv_sem=recv_sems.at[slot, g],
                device_id=(0,),
                device_id_type=pl.DeviceIdType.MESH,
            ).wait_recv()

        wob = wo_ref[...].astype(jnp.bfloat16)
        for g in range(GQ):
            @pl.when(my != 0)
            def _():
                wait_recv(0, g)

            @pl.when(my != 2)
            def _():
                wait_recv(1, g)

            for b in range(B):
                ctx_heads = []
                for h in range(HQ):
                    p0 = cl_buf[0, b, g, h].astype(jnp.float32)
                    p2 = cl_buf[1, b, g, h].astype(jnp.float32)
                    p = p0 + p2
                    ctx_heads.append(p[:, :DH] / p[:, DH:PK])
                ctx = jnp.concatenate(ctx_heads, axis=1).astype(jnp.bfloat16)
                out_ref[b, pl.ds(g * BLK, BLK), :] = jnp.dot(
                    ctx, wob, preferred_element_type=jnp.float32)

        @pl.when((my == 0) | (my == 2))
        def _():
            for j in range(6):
                pltpu.make_async_remote_copy(
                    src_ref=cl_buf.at[0, :, 0],
                    dst_ref=cl_buf.at[0, :, 0],
                    send_sem=send_sems.at[j],
                    recv_sem=recv_sems.at[0, 0],
                    device_id=(0,),
                    device_id_type=pl.DeviceIdType.MESH,
                ).wait_send()

    return pl.pallas_call(
        body,
        out_shape=jax.ShapeDtypeStruct((B, SQ, DM), jnp.float32),
        in_specs=[pl.BlockSpec(memory_space=pltpu.VMEM)] * 5,
        out_specs=pl.BlockSpec(memory_space=pltpu.VMEM),
        scratch_shapes=[
            pltpu.VMEM((2, B, GQ, HQ, BLK, PK), jnp.bfloat16),
            pltpu.SemaphoreType.DMA((6,)),
            pltpu.SemaphoreType.DMA((2, 2)),
        ],
        compiler_params=pltpu.CompilerParams(collective_id=0),
    )(x, Wq, K_ext, V_ext, Wo)
